# baseline (speedup 1.0000x reference)
import jax
import jax.numpy as jnp
from jax import lax
from jax.experimental import pallas as pl
from jax.experimental.pallas import tpu as pltpu
from jax.experimental.pallas import tpu_sc as plsc

N = 10000
G = 512
E = 320000
NP = 10240
EPT = E // 32
CH = 128
NFULL = EPT // CH
TAIL = EPT - NFULL * CH
NC, NS = 2, 16
RPT = NP // NS
BLK = 256
NBLK = NP // BLK


def _sc_mesh():
    return plsc.VectorSubcoreMesh(core_axis_name="c", subcore_axis_name="s")


DB = 3


def _deg_full_body(dst_hbm, batch_hbm, deg_hbm, cnt_hbm, deg_sh, cnt_sh,
                   idx0, idx1, idx2, tail_i, bidx, ones_v, half_v, work_v,
                   ss0, ss1, ss2):
    c = lax.axis_index("c")
    s = lax.axis_index("s")
    idxs = (idx0, idx1, idx2)
    sss = (ss0, ss1, ss2)

    def fill(i, _):
        ones_v[pl.ds(i * 16, 16)] = jnp.full((16,), 1.0, jnp.float32)
        half_v[pl.ds(i * 16, 16)] = jnp.full((16,), 0.5, jnp.float32)
        work_v[pl.ds(i * 16, 16)] = jnp.zeros((16,), jnp.float32)
        return 0
    lax.fori_loop(0, CH // 16, fill, 0)

    base = s * RPT
    for k in range(RPT // CH):
        pltpu.sync_copy(half_v, deg_sh.at[pl.ds(base + k * CH, CH)])
    pltpu.sync_copy(work_v.at[pl.ds(0, 64)], cnt_sh.at[pl.ds(s * 64, 64)])
    plsc.subcore_barrier()

    estart = (c * NS + s) * EPT

    def body(t, _):
        for b in range(DB):
            @pl.when(t > 0)
            def _(b=b):
                pltpu.make_async_copy(ones_v, deg_sh.at[idxs[b]], sss[b]).wait()
            pltpu.sync_copy(dst_hbm.at[pl.ds(estart + (DB * t + b) * CH, CH)],
                            idxs[b])
            pltpu.async_copy(ones_v, deg_sh.at[idxs[b]], sss[b], add=True)
        return 0
    lax.fori_loop(0, NFULL // DB, body, 0)
    for b in range(DB):
        pltpu.make_async_copy(ones_v, deg_sh.at[idxs[b]], sss[b]).wait()
    pltpu.sync_copy(dst_hbm.at[pl.ds(estart + NFULL * CH, TAIL)], tail_i)
    pltpu.sync_copy(ones_v.at[pl.ds(0, TAIL)], deg_sh.at[tail_i], add=True)

    nb = c * (NP // NC) + s * (NP // NC // NS)
    pltpu.sync_copy(batch_hbm.at[pl.ds(nb, CH)], idx0)
    pltpu.sync_copy(ones_v, cnt_sh.at[idx0], add=True)
    pltpu.sync_copy(batch_hbm.at[pl.ds(nb + CH, CH)], idx1)
    pltpu.sync_copy(ones_v, cnt_sh.at[idx1], add=True)
    pltpu.sync_copy(batch_hbm.at[pl.ds(nb + 2 * CH, 64)], bidx)
    pltpu.sync_copy(ones_v.at[pl.ds(0, 64)], cnt_sh.at[bidx], add=True)
    plsc.subcore_barrier()

    pltpu.sync_copy(deg_sh.at[pl.ds(base, RPT)], work_v)
    pltpu.sync_copy(work_v, deg_hbm.at[pl.ds(c * NP + base, RPT)])
    pltpu.sync_copy(cnt_sh.at[pl.ds(s * 32, 32)], work_v.at[pl.ds(0, 32)])
    pltpu.sync_copy(work_v.at[pl.ds(0, 32)],
                    cnt_hbm.at[pl.ds(c * G + s * 32, 32)])


def _deg(dst, batch_p):
    k = pl.kernel(
        _deg_full_body,
        out_type=(jax.ShapeDtypeStruct((NC * NP,), jnp.float32),
                  jax.ShapeDtypeStruct((NC * G,), jnp.float32)),
        mesh=_sc_mesh(),
        scratch_types=[
            pltpu.VMEM_SHARED((NP,), jnp.float32),
            pltpu.VMEM_SHARED((1024,), jnp.float32),
            pltpu.VMEM((CH,), jnp.int32),
            pltpu.VMEM((CH,), jnp.int32),
            pltpu.VMEM((CH,), jnp.int32),
            pltpu.VMEM((TAIL,), jnp.int32),
            pltpu.VMEM((64,), jnp.int32),
            pltpu.VMEM((CH,), jnp.float32),
            pltpu.VMEM((CH,), jnp.float32),
            pltpu.VMEM((RPT,), jnp.float32),
            pltpu.SemaphoreType.DMA,
            pltpu.SemaphoreType.DMA,
            pltpu.SemaphoreType.DMA,
        ],
    )
    return k(dst, batch_p)


SB = 3
AR = 10112
ARPT = AR // NS


def _scatter_body(h_hbm, src_hbm, dst_hbm, out_hbm, acc_sh, *rest):
    srcs = rest[0:SB]
    dsts = rest[SB:2 * SB]
    rows = rest[2 * SB:3 * SB]
    tsrc, tdst = rest[3 * SB:3 * SB + 2]
    gss = rest[3 * SB + 2:4 * SB + 2]
    sss = rest[4 * SB + 2:5 * SB + 2]
    iss = rest[5 * SB + 2:6 * SB + 2]
    ids = rest[6 * SB + 2:7 * SB + 2]
    tsem = rest[7 * SB + 2]
    c = lax.axis_index("c")
    s = lax.axis_index("s")

    def zr(r, _):
        for k in range(8):
            rows[0][r, pl.ds(k * 16, 16)] = jnp.zeros((16,), jnp.float32)
        return 0
    lax.fori_loop(0, CH, zr, 0)

    base = s * ARPT
    for k in range(ARPT // CH):
        pltpu.sync_copy(rows[0], acc_sh.at[pl.ds(base + k * CH, CH)])
    pltpu.sync_copy(rows[0].at[pl.ds(0, ARPT % CH)],
                    acc_sh.at[pl.ds(base + (ARPT // CH) * CH, ARPT % CH)])
    plsc.subcore_barrier()

    estart = (c * NS + s) * EPT

    def body(t, _):
        for b in range(SB):
            @pl.when(t > 0)
            def _(b=b):
                pltpu.make_async_copy(rows[b], acc_sh.at[dsts[b]], sss[b]).wait()
            bo = estart + (SB * t + b) * CH
            pltpu.async_copy(src_hbm.at[pl.ds(bo, CH)], srcs[b], iss[b])
            pltpu.async_copy(dst_hbm.at[pl.ds(bo, CH)], dsts[b], ids[b])
        for b in range(SB):
            bo = estart + (SB * t + b) * CH
            pltpu.make_async_copy(src_hbm.at[pl.ds(bo, CH)], srcs[b],
                                  iss[b]).wait()
            pltpu.async_copy(h_hbm.at[srcs[b]], rows[b], gss[b])
        for b in range(SB):
            bo = estart + (SB * t + b) * CH
            pltpu.make_async_copy(h_hbm.at[srcs[b]], rows[b], gss[b]).wait()
            pltpu.make_async_copy(dst_hbm.at[pl.ds(bo, CH)], dsts[b],
                                  ids[b]).wait()
            pltpu.async_copy(rows[b], acc_sh.at[dsts[b]], sss[b], add=True)
        return 0
    lax.fori_loop(0, NFULL // SB, body, 0)
    for b in range(SB):
        pltpu.make_async_copy(rows[b], acc_sh.at[dsts[b]], sss[b]).wait()

    tb = estart + NFULL * CH
    pltpu.sync_copy(src_hbm.at[pl.ds(tb, TAIL)], tsrc)
    pltpu.sync_copy(dst_hbm.at[pl.ds(tb, TAIL)], tdst)
    pltpu.async_copy(h_hbm.at[tsrc], rows[0].at[pl.ds(0, TAIL)], tsem).wait()
    pltpu.sync_copy(rows[0].at[pl.ds(0, TAIL)], acc_sh.at[tdst], add=True)
    plsc.subcore_barrier()

    ob = c * NP + base
    pltpu.sync_copy(acc_sh.at[pl.ds(base, ARPT)],
                    out_hbm.at[pl.ds(ob, ARPT)])


def _edge_scatter(h, src, dst):
    k = pl.kernel(
        _scatter_body,
        out_type=jax.ShapeDtypeStruct((NC * NP, 128), jnp.float32),
        mesh=_sc_mesh(),
        scratch_types=(
            [pltpu.VMEM_SHARED((AR, 128), jnp.float32)]
            + [pltpu.VMEM((CH,), jnp.int32) for _ in range(2 * SB)]
            + [pltpu.VMEM((CH, 128), jnp.float32) for _ in range(SB)]
            + [pltpu.VMEM((TAIL,), jnp.int32) for _ in range(2)]
            + [pltpu.SemaphoreType.DMA for _ in range(4 * SB + 1)]
        ),
    )
    return k(h, src, dst)


def _mm_scale_body(x_ref, w_ref, d0_ref, d1_ref, o_ref, dv_ref):
    dinv = lax.rsqrt(d0_ref[...] + d1_ref[...])
    dv_ref[...] = dinv
    o_ref[...] = jnp.dot(x_ref[...], w_ref[...],
                         preferred_element_type=jnp.float32) * dinv


def _mm_scale(x_p, W, deg2):
    return pl.pallas_call(
        _mm_scale_body,
        out_shape=(jax.ShapeDtypeStruct((NP, 128), jnp.float32),
                   jax.ShapeDtypeStruct((NP, 1), jnp.float32)),
        grid=(NBLK,),
        in_specs=[pl.BlockSpec((BLK, 128), lambda i: (i, 0)),
                  pl.BlockSpec((128, 128), lambda i: (0, 0)),
                  pl.BlockSpec((BLK, 1), lambda i: (i, 0)),
                  pl.BlockSpec((BLK, 1), lambda i: (i + NBLK, 0))],
        out_specs=(pl.BlockSpec((BLK, 128), lambda i: (i, 0)),
                   pl.BlockSpec((BLK, 1), lambda i: (i, 0))),
    )(x_p, W, deg2, deg2)


def _layer_mid_body(s0_ref, s1_ref, h_ref, d_ref, b_ref, w_ref, o_ref):
    i = pl.program_id(0)
    tot = s0_ref[...] + s1_ref[...] + h_ref[...]
    act = jnp.maximum(d_ref[...] * tot + b_ref[...], 0.0)
    row = i * BLK + lax.broadcasted_iota(jnp.int32, (BLK, 128), 0)
    act = jnp.where(row < N, act, 0.0)
    o_ref[...] = jnp.dot(act, w_ref[...],
                         preferred_element_type=jnp.float32) * d_ref[...]


def _layer_mid(s, ht, dinv2, b1r, W2):
    return pl.pallas_call(
        _layer_mid_body,
        out_shape=jax.ShapeDtypeStruct((NP, 128), jnp.float32),
        grid=(NBLK,),
        in_specs=[pl.BlockSpec((BLK, 128), lambda i: (i, 0)),
                  pl.BlockSpec((BLK, 128), lambda i: (i + NBLK, 0)),
                  pl.BlockSpec((BLK, 128), lambda i: (i, 0)),
                  pl.BlockSpec((BLK, 1), lambda i: (i, 0)),
                  pl.BlockSpec((1, 128), lambda i: (0, 0)),
                  pl.BlockSpec((128, 128), lambda i: (0, 0))],
        out_specs=pl.BlockSpec((BLK, 128), lambda i: (i, 0)),
    )(s, s, ht, dinv2, b1r, W2)


def _finale_body(s0_ref, s1_ref, h_ref, d_ref, b_ref, bt_ref, c0_ref, c1_ref,
                 wl_ref, bl_ref, wd1_ref, bd1_ref, wd2_ref, bd2_ref, o_ref):
    i = pl.program_id(0)
    tot = s0_ref[...] + s1_ref[...] + h_ref[...]
    act = jnp.maximum(d_ref[...] * tot + b_ref[...], 0.0)
    row = i * BLK + lax.broadcasted_iota(jnp.int32, (BLK, 128), 0)
    act = jnp.where(row < N, act, 0.0)
    gid = lax.broadcasted_iota(jnp.int32, (BLK, G), 1)
    onehot = jnp.where(bt_ref[...].astype(jnp.int32) == gid, 1.0, 0.0)
    part = lax.dot_general(onehot, act, (((0,), (0,)), ((), ())),
                           preferred_element_type=jnp.float32)

    @pl.when(i == 0)
    def _():
        o_ref[...] = jnp.zeros_like(o_ref)

    o_ref[...] += part

    @pl.when(i == NBLK - 1)
    def _():
        pooled = o_ref[...] / jnp.maximum(c0_ref[...] + c1_ref[...], 1.0)
        lab = jax.nn.sigmoid(
            jnp.dot(pooled, wl_ref[...], preferred_element_type=jnp.float32)
            + bl_ref[...])
        dmid = jnp.maximum(
            jnp.dot(pooled, wd1_ref[...], preferred_element_type=jnp.float32)
            + bd1_ref[...], 0.0)
        dom = (jnp.dot(dmid, wd2_ref[...], preferred_element_type=jnp.float32)
               + bd2_ref[...])
        col = lax.broadcasted_iota(jnp.int32, (G, 128), 1)
        o_ref[...] = jnp.where(col == 0, lab, dom)


def _finale(s, ht, dinv2, b2r, batchf, cnt2, Wlp, blp, Wd1, bd1r, Wd2p, bd2p):
    return pl.pallas_call(
        _finale_body,
        out_shape=jax.ShapeDtypeStruct((G, 128), jnp.float32),
        grid=(NBLK,),
        in_specs=[pl.BlockSpec((BLK, 128), lambda i: (i, 0)),
                  pl.BlockSpec((BLK, 128), lambda i: (i + NBLK, 0)),
                  pl.BlockSpec((BLK, 128), lambda i: (i, 0)),
                  pl.BlockSpec((BLK, 1), lambda i: (i, 0)),
                  pl.BlockSpec((1, 128), lambda i: (0, 0)),
                  pl.BlockSpec((BLK, 1), lambda i: (i, 0)),
                  pl.BlockSpec((G, 1), lambda i: (0, 0)),
                  pl.BlockSpec((G, 1), lambda i: (1, 0)),
                  pl.BlockSpec((128, 128), lambda i: (0, 0)),
                  pl.BlockSpec((1, 128), lambda i: (0, 0)),
                  pl.BlockSpec((128, 64), lambda i: (0, 0)),
                  pl.BlockSpec((1, 64), lambda i: (0, 0)),
                  pl.BlockSpec((64, 128), lambda i: (0, 0)),
                  pl.BlockSpec((1, 128), lambda i: (0, 0))],
        out_specs=pl.BlockSpec((G, 128), lambda i: (0, 0)),
    )(s, s, ht, dinv2, b2r, batchf, cnt2, cnt2, Wlp, blp, Wd1, bd1r, Wd2p, bd2p)


def kernel(x, edge_index, batch, W1, b1, W2, b2, Wl, bl, Wd1, bd1, Wd2, bd2):
    npad = NP - N
    src = edge_index[0]
    dst = edge_index[1]
    batch_p = jnp.concatenate([batch, jnp.full((npad,), G, jnp.int32)])
    x_p = jnp.pad(x, ((0, npad), (0, 0)))

    deg, cnt = _deg(dst, batch_p)
    h1t, dinv2 = _mm_scale(x_p, W1, deg.reshape(NC * NP, 1))
    s1 = _edge_scatter(h1t, src, dst)
    h2t = _layer_mid(s1, h1t, dinv2, b1[None, :], W2)
    s2 = _edge_scatter(h2t, src, dst)

    heads = _finale(
        s2, h2t, dinv2, b2[None, :],
        batch_p.astype(jnp.float32).reshape(NP, 1), cnt.reshape(NC * G, 1),
        jnp.pad(Wl, ((0, 0), (0, 127))), jnp.pad(bl[None, :], ((0, 0), (0, 127))),
        Wd1, bd1[None, :],
        jnp.pad(Wd2, ((0, 0), (1, 125))), jnp.pad(bd2[None, :], ((0, 0), (1, 125))),
    )
    return heads[:, 0:1], heads[:, 1:3]


# speedup vs baseline: 1.0303x; 1.0303x over previous
import jax
import jax.numpy as jnp
from jax import lax
from jax.experimental import pallas as pl
from jax.experimental.pallas import tpu as pltpu
from jax.experimental.pallas import tpu_sc as plsc

N = 10000
G = 512
E = 320000
NP = 10240
CH = 128
ECH = 2560
CPT = ECH // 32
NST = 2
SCH = CPT // NST
NC, NS = 2, 16
RPT = NP // NS
BLK = 256
NBLK = NP // BLK


def _sc_mesh():
    return plsc.VectorSubcoreMesh(core_axis_name="c", subcore_axis_name="s")


DB = 3


def _deg_full_body(dst_hbm, batch_hbm, deg_hbm, cnt_hbm, deg_sh, cnt_sh,
                   dstage, bidx0, bidx1, bidx2, ones_v, half_v, work_v,
                   ss0, ss1):
    c = lax.axis_index("c")
    s = lax.axis_index("s")
    sss = (ss0, ss1)

    def fill(i, _):
        ones_v[pl.ds(i * 16, 16)] = jnp.full((16,), 1.0, jnp.float32)
        half_v[pl.ds(i * 16, 16)] = jnp.full((16,), 0.5, jnp.float32)
        work_v[pl.ds(i * 16, 16)] = jnp.zeros((16,), jnp.float32)
        return 0
    lax.fori_loop(0, CH // 16, fill, 0)

    base = s * RPT
    for k in range(RPT // CH):
        pltpu.sync_copy(half_v, deg_sh.at[pl.ds(base + k * CH, CH)])
    pltpu.sync_copy(work_v.at[pl.ds(0, 64)], cnt_sh.at[pl.ds(s * 64, 64)])
    plsc.subcore_barrier()

    crow = (c * NS + s) * CPT
    for st in range(NST):
        pltpu.sync_copy(dst_hbm.at[pl.ds(crow + st * SCH, SCH)], dstage)

        def body(r, _):
            for b in range(2):
                @pl.when(r > 0)
                def _(b=b):
                    pltpu.make_async_copy(
                        ones_v, deg_sh.at[dstage.at[0]], sss[b]).wait()
                pltpu.async_copy(ones_v, deg_sh.at[dstage.at[2 * r + b]],
                                 sss[b], add=True)
            return 0
        lax.fori_loop(0, SCH // 2, body, 0)
        for b in range(2):
            pltpu.make_async_copy(ones_v, deg_sh.at[dstage.at[0]],
                                  sss[b]).wait()

    nb = c * (NP // NC) + s * (NP // NC // NS)
    pltpu.sync_copy(batch_hbm.at[pl.ds(nb, CH)], bidx0)
    pltpu.sync_copy(ones_v, cnt_sh.at[bidx0], add=True)
    pltpu.sync_copy(batch_hbm.at[pl.ds(nb + CH, CH)], bidx1)
    pltpu.sync_copy(ones_v, cnt_sh.at[bidx1], add=True)
    pltpu.sync_copy(batch_hbm.at[pl.ds(nb + 2 * CH, 64)], bidx2)
    pltpu.sync_copy(ones_v.at[pl.ds(0, 64)], cnt_sh.at[bidx2], add=True)
    plsc.subcore_barrier()

    pltpu.sync_copy(deg_sh.at[pl.ds(base, RPT)], work_v)
    pltpu.sync_copy(work_v, deg_hbm.at[pl.ds(c * NP + base, RPT)])
    pltpu.sync_copy(cnt_sh.at[pl.ds(s * 32, 32)], work_v.at[pl.ds(0, 32)])
    pltpu.sync_copy(work_v.at[pl.ds(0, 32)],
                    cnt_hbm.at[pl.ds(c * G + s * 32, 32)])


def _deg(dst2, batch_p):
    k = pl.kernel(
        _deg_full_body,
        out_type=(jax.ShapeDtypeStruct((NC * NP,), jnp.float32),
                  jax.ShapeDtypeStruct((NC * G,), jnp.float32)),
        mesh=_sc_mesh(),
        scratch_types=[
            pltpu.VMEM_SHARED((NP,), jnp.float32),
            pltpu.VMEM_SHARED((1024,), jnp.float32),
            pltpu.VMEM((SCH, CH), jnp.int32),
            pltpu.VMEM((CH,), jnp.int32),
            pltpu.VMEM((CH,), jnp.int32),
            pltpu.VMEM((64,), jnp.int32),
            pltpu.VMEM((CH,), jnp.float32),
            pltpu.VMEM((CH,), jnp.float32),
            pltpu.VMEM((RPT,), jnp.float32),
            pltpu.SemaphoreType.DMA,
            pltpu.SemaphoreType.DMA,
        ],
    )
    return k(dst2, batch_p)


SB = 2
AR = 10112
ARPT = AR // NS


def _scatter_body(h_hbm, src_hbm, dst_hbm, out_hbm, acc_sh,
                  sstage, dstage, rows0, rows1, gs0, gs1, ss0, ss1):
    rows = (rows0, rows1)
    gss = (gs0, gs1)
    sss = (ss0, ss1)
    c = lax.axis_index("c")
    s = lax.axis_index("s")

    def zr(r, _):
        for k in range(8):
            rows0[r, pl.ds(k * 16, 16)] = jnp.zeros((16,), jnp.float32)
        return 0
    lax.fori_loop(0, CH, zr, 0)

    base = s * ARPT
    for k in range(ARPT // CH):
        pltpu.sync_copy(rows0, acc_sh.at[pl.ds(base + k * CH, CH)])
    pltpu.sync_copy(rows0.at[pl.ds(0, ARPT % CH)],
                    acc_sh.at[pl.ds(base + (ARPT // CH) * CH, ARPT % CH)])
    plsc.subcore_barrier()

    crow = (c * NS + s) * CPT

    for st in range(NST):
        pltpu.sync_copy(src_hbm.at[pl.ds(crow + st * SCH, SCH)], sstage)
        pltpu.sync_copy(dst_hbm.at[pl.ds(crow + st * SCH, SCH)], dstage)

        def body(r, _):
            for b in range(2):
                j = 2 * r + b

                @pl.when(r > 0)
                def _(b=b):
                    pltpu.make_async_copy(rows[b], acc_sh.at[dstage.at[0]],
                                          sss[b]).wait()
                pltpu.async_copy(h_hbm.at[sstage.at[j]], rows[b], gss[b])
            for b in range(2):
                j = 2 * r + b
                pltpu.make_async_copy(h_hbm.at[sstage.at[j]], rows[b],
                                      gss[b]).wait()
                pltpu.async_copy(rows[b], acc_sh.at[dstage.at[j]], sss[b],
                                 add=True)
            return 0
        lax.fori_loop(0, SCH // 2, body, 0)
        for b in range(2):
            pltpu.make_async_copy(rows[b], acc_sh.at[dstage.at[0]],
                                  sss[b]).wait()
    plsc.subcore_barrier()

    ob = c * NP + base
    pltpu.sync_copy(acc_sh.at[pl.ds(base, ARPT)],
                    out_hbm.at[pl.ds(ob, ARPT)])


def _edge_scatter(h, src2, dst2):
    k = pl.kernel(
        _scatter_body,
        out_type=jax.ShapeDtypeStruct((NC * NP, 128), jnp.float32),
        mesh=_sc_mesh(),
        scratch_types=(
            [pltpu.VMEM_SHARED((AR, 128), jnp.float32)]
            + [pltpu.VMEM((SCH, CH), jnp.int32) for _ in range(2)]
            + [pltpu.VMEM((CH, 128), jnp.float32) for _ in range(2)]
            + [pltpu.SemaphoreType.DMA for _ in range(4)]
        ),
    )
    return k(h, src2, dst2)


def _mm_scale_body(x_ref, w_ref, d0_ref, d1_ref, o_ref, dv_ref):
    dinv = lax.rsqrt(d0_ref[...] + d1_ref[...])
    dv_ref[...] = dinv
    o_ref[...] = jnp.dot(x_ref[...], w_ref[...],
                         preferred_element_type=jnp.float32) * dinv


def _mm_scale(x_p, W, deg2):
    return pl.pallas_call(
        _mm_scale_body,
        out_shape=(jax.ShapeDtypeStruct((NP, 128), jnp.float32),
                   jax.ShapeDtypeStruct((NP, 1), jnp.float32)),
        grid=(NBLK,),
        in_specs=[pl.BlockSpec((BLK, 128), lambda i: (i, 0)),
                  pl.BlockSpec((128, 128), lambda i: (0, 0)),
                  pl.BlockSpec((BLK, 1), lambda i: (i, 0)),
                  pl.BlockSpec((BLK, 1), lambda i: (i + NBLK, 0))],
        out_specs=(pl.BlockSpec((BLK, 128), lambda i: (i, 0)),
                   pl.BlockSpec((BLK, 1), lambda i: (i, 0))),
    )(x_p, W, deg2, deg2)


def _layer_mid_body(s0_ref, s1_ref, h_ref, d_ref, b_ref, w_ref, o_ref):
    i = pl.program_id(0)
    tot = s0_ref[...] + s1_ref[...] + h_ref[...]
    act = jnp.maximum(d_ref[...] * tot + b_ref[...], 0.0)
    row = i * BLK + lax.broadcasted_iota(jnp.int32, (BLK, 128), 0)
    act = jnp.where(row < N, act, 0.0)
    o_ref[...] = jnp.dot(act, w_ref[...],
                         preferred_element_type=jnp.float32) * d_ref[...]


def _layer_mid(s, ht, dinv2, b1r, W2):
    return pl.pallas_call(
        _layer_mid_body,
        out_shape=jax.ShapeDtypeStruct((NP, 128), jnp.float32),
        grid=(NBLK,),
        in_specs=[pl.BlockSpec((BLK, 128), lambda i: (i, 0)),
                  pl.BlockSpec((BLK, 128), lambda i: (i + NBLK, 0)),
                  pl.BlockSpec((BLK, 128), lambda i: (i, 0)),
                  pl.BlockSpec((BLK, 1), lambda i: (i, 0)),
                  pl.BlockSpec((1, 128), lambda i: (0, 0)),
                  pl.BlockSpec((128, 128), lambda i: (0, 0))],
        out_specs=pl.BlockSpec((BLK, 128), lambda i: (i, 0)),
    )(s, s, ht, dinv2, b1r, W2)


def _finale_body(s0_ref, s1_ref, h_ref, d_ref, b_ref, bt_ref, c0_ref, c1_ref,
                 wl_ref, bl_ref, wd1_ref, bd1_ref, wd2_ref, bd2_ref, o_ref):
    i = pl.program_id(0)
    tot = s0_ref[...] + s1_ref[...] + h_ref[...]
    act = jnp.maximum(d_ref[...] * tot + b_ref[...], 0.0)
    row = i * BLK + lax.broadcasted_iota(jnp.int32, (BLK, 128), 0)
    act = jnp.where(row < N, act, 0.0)
    gid = lax.broadcasted_iota(jnp.int32, (BLK, G), 1)
    onehot = jnp.where(bt_ref[...].astype(jnp.int32) == gid, 1.0, 0.0)
    part = lax.dot_general(onehot, act, (((0,), (0,)), ((), ())),
                           preferred_element_type=jnp.float32)

    @pl.when(i == 0)
    def _():
        o_ref[...] = jnp.zeros_like(o_ref)

    o_ref[...] += part

    @pl.when(i == NBLK - 1)
    def _():
        pooled = o_ref[...] / jnp.maximum(c0_ref[...] + c1_ref[...], 1.0)
        lab = jax.nn.sigmoid(
            jnp.dot(pooled, wl_ref[...], preferred_element_type=jnp.float32)
            + bl_ref[...])
        dmid = jnp.maximum(
            jnp.dot(pooled, wd1_ref[...], preferred_element_type=jnp.float32)
            + bd1_ref[...], 0.0)
        dom = (jnp.dot(dmid, wd2_ref[...], preferred_element_type=jnp.float32)
               + bd2_ref[...])
        col = lax.broadcasted_iota(jnp.int32, (G, 128), 1)
        o_ref[...] = jnp.where(col == 0, lab, dom)


def _finale(s, ht, dinv2, b2r, batchf, cnt2, Wlp, blp, Wd1, bd1r, Wd2p, bd2p):
    return pl.pallas_call(
        _finale_body,
        out_shape=jax.ShapeDtypeStruct((G, 128), jnp.float32),
        grid=(NBLK,),
        in_specs=[pl.BlockSpec((BLK, 128), lambda i: (i, 0)),
                  pl.BlockSpec((BLK, 128), lambda i: (i + NBLK, 0)),
                  pl.BlockSpec((BLK, 128), lambda i: (i, 0)),
                  pl.BlockSpec((BLK, 1), lambda i: (i, 0)),
                  pl.BlockSpec((1, 128), lambda i: (0, 0)),
                  pl.BlockSpec((BLK, 1), lambda i: (i, 0)),
                  pl.BlockSpec((G, 1), lambda i: (0, 0)),
                  pl.BlockSpec((G, 1), lambda i: (1, 0)),
                  pl.BlockSpec((128, 128), lambda i: (0, 0)),
                  pl.BlockSpec((1, 128), lambda i: (0, 0)),
                  pl.BlockSpec((128, 64), lambda i: (0, 0)),
                  pl.BlockSpec((1, 64), lambda i: (0, 0)),
                  pl.BlockSpec((64, 128), lambda i: (0, 0)),
                  pl.BlockSpec((1, 128), lambda i: (0, 0))],
        out_specs=pl.BlockSpec((G, 128), lambda i: (0, 0)),
    )(s, s, ht, dinv2, b2r, batchf, cnt2, cnt2, Wlp, blp, Wd1, bd1r, Wd2p, bd2p)


def kernel(x, edge_index, batch, W1, b1, W2, b2, Wl, bl, Wd1, bd1, Wd2, bd2):
    npad = NP - N
    epad = ECH * CH - E
    ar = jnp.arange(epad, dtype=jnp.int32)
    src2 = jnp.concatenate([edge_index[0], N + ar % npad]).reshape(ECH, CH)
    dst2 = jnp.concatenate([edge_index[1], N + ar % (AR - N)]).reshape(ECH, CH)
    batch_p = jnp.concatenate([batch, jnp.full((npad,), G, jnp.int32)])
    x_p = jnp.pad(x, ((0, npad), (0, 0)))

    deg, cnt = _deg(dst2, batch_p)
    h1t, dinv2 = _mm_scale(x_p, W1, deg.reshape(NC * NP, 1))
    s1 = _edge_scatter(h1t, src2, dst2)
    h2t = _layer_mid(s1, h1t, dinv2, b1[None, :], W2)
    s2 = _edge_scatter(h2t, src2, dst2)

    heads = _finale(
        s2, h2t, dinv2, b2[None, :],
        batch_p.astype(jnp.float32).reshape(NP, 1), cnt.reshape(NC * G, 1),
        jnp.pad(Wl, ((0, 0), (0, 127))), jnp.pad(bl[None, :], ((0, 0), (0, 127))),
        Wd1, bd1[None, :],
        jnp.pad(Wd2, ((0, 0), (1, 125))), jnp.pad(bd2[None, :], ((0, 0), (1, 125))),
    )
    return heads[:, 0:1], heads[:, 1:3]


# speedup vs baseline: 1.0335x; 1.0031x over previous
import jax
import jax.numpy as jnp
from jax import lax
from jax.experimental import pallas as pl
from jax.experimental.pallas import tpu as pltpu
from jax.experimental.pallas import tpu_sc as plsc

N = 10000
G = 512
E = 320000
NP = 10240
CH = 128
ECH = 2560
CPT = ECH // 32
NST = 2
SCH = CPT // NST
NC, NS = 2, 16
RPT = NP // NS
BLK = 256
NBLK = NP // BLK


def _sc_mesh():
    return plsc.VectorSubcoreMesh(core_axis_name="c", subcore_axis_name="s")


DB = 3


def _deg_full_body(dst_hbm, batch_hbm, deg_hbm, cnt_hbm, deg_sh, cnt_sh,
                   dstage, bidx0, bidx1, bidx2, ones_v, half_v, work_v,
                   ss0, ss1):
    c = lax.axis_index("c")
    s = lax.axis_index("s")
    sss = (ss0, ss1)

    def fill(i, _):
        ones_v[pl.ds(i * 16, 16)] = jnp.full((16,), 1.0, jnp.float32)
        half_v[pl.ds(i * 16, 16)] = jnp.full((16,), 0.5, jnp.float32)
        work_v[pl.ds(i * 16, 16)] = jnp.zeros((16,), jnp.float32)
        return 0
    lax.fori_loop(0, CH // 16, fill, 0)

    base = s * RPT
    for k in range(RPT // CH):
        pltpu.sync_copy(half_v, deg_sh.at[pl.ds(base + k * CH, CH)])
    pltpu.sync_copy(work_v.at[pl.ds(0, 64)], cnt_sh.at[pl.ds(s * 64, 64)])
    plsc.subcore_barrier()

    crow = (c * NS + s) * CPT
    for st in range(NST):
        pltpu.sync_copy(dst_hbm.at[pl.ds(crow + st * SCH, SCH)], dstage)

        def body(r, _):
            for b in range(2):
                @pl.when(r > 0)
                def _(b=b):
                    pltpu.make_async_copy(
                        ones_v, deg_sh.at[dstage.at[0]], sss[b]).wait()
                pltpu.async_copy(ones_v, deg_sh.at[dstage.at[2 * r + b]],
                                 sss[b], add=True)
            return 0
        lax.fori_loop(0, SCH // 2, body, 0)
        for b in range(2):
            pltpu.make_async_copy(ones_v, deg_sh.at[dstage.at[0]],
                                  sss[b]).wait()

    nb = c * (NP // NC) + s * (NP // NC // NS)
    pltpu.sync_copy(batch_hbm.at[pl.ds(nb, CH)], bidx0)
    pltpu.sync_copy(ones_v, cnt_sh.at[bidx0], add=True)
    pltpu.sync_copy(batch_hbm.at[pl.ds(nb + CH, CH)], bidx1)
    pltpu.sync_copy(ones_v, cnt_sh.at[bidx1], add=True)
    pltpu.sync_copy(batch_hbm.at[pl.ds(nb + 2 * CH, 64)], bidx2)
    pltpu.sync_copy(ones_v.at[pl.ds(0, 64)], cnt_sh.at[bidx2], add=True)
    plsc.subcore_barrier()

    pltpu.sync_copy(deg_sh.at[pl.ds(base, RPT)], work_v)
    pltpu.sync_copy(work_v, deg_hbm.at[pl.ds(c * NP + base, RPT)])
    pltpu.sync_copy(cnt_sh.at[pl.ds(s * 32, 32)], work_v.at[pl.ds(0, 32)])
    pltpu.sync_copy(work_v.at[pl.ds(0, 32)],
                    cnt_hbm.at[pl.ds(c * G + s * 32, 32)])


def _deg(dst2, batch_p):
    k = pl.kernel(
        _deg_full_body,
        out_type=(jax.ShapeDtypeStruct((NC * NP,), jnp.float32),
                  jax.ShapeDtypeStruct((NC * G,), jnp.float32)),
        mesh=_sc_mesh(),
        scratch_types=[
            pltpu.VMEM_SHARED((NP,), jnp.float32),
            pltpu.VMEM_SHARED((1024,), jnp.float32),
            pltpu.VMEM((SCH, CH), jnp.int32),
            pltpu.VMEM((CH,), jnp.int32),
            pltpu.VMEM((CH,), jnp.int32),
            pltpu.VMEM((64,), jnp.int32),
            pltpu.VMEM((CH,), jnp.float32),
            pltpu.VMEM((CH,), jnp.float32),
            pltpu.VMEM((RPT,), jnp.float32),
            pltpu.SemaphoreType.DMA,
            pltpu.SemaphoreType.DMA,
        ],
    )
    return k(dst2, batch_p)


SB = 2
AR = 10112
ARPT = AR // NS


def _scatter_body(h_hbm, src_hbm, dst_hbm, out_hbm, acc_sh,
                  sstage, dstage, rows0, rows1, gs0, gs1, ss0, ss1):
    rows = (rows0, rows1)
    gss = (gs0, gs1)
    sss = (ss0, ss1)
    c = lax.axis_index("c")
    s = lax.axis_index("s")

    def zr(r, _):
        for k in range(8):
            rows0[r, pl.ds(k * 16, 16)] = jnp.zeros((16,), jnp.float32)
        return 0
    lax.fori_loop(0, CH, zr, 0)

    base = s * ARPT
    for k in range(ARPT // CH):
        pltpu.sync_copy(rows0, acc_sh.at[pl.ds(base + k * CH, CH)])
    pltpu.sync_copy(rows0.at[pl.ds(0, ARPT % CH)],
                    acc_sh.at[pl.ds(base + (ARPT // CH) * CH, ARPT % CH)])
    plsc.subcore_barrier()

    crow = (c * NS + s) * CPT

    for st in range(NST):
        pltpu.sync_copy(src_hbm.at[pl.ds(crow + st * SCH, SCH)], sstage)
        pltpu.sync_copy(dst_hbm.at[pl.ds(crow + st * SCH, SCH)], dstage)

        def body(r, _):
            for b in range(2):
                j = 2 * r + b

                @pl.when(r > 0)
                def _(b=b):
                    pltpu.make_async_copy(rows[b], acc_sh.at[dstage.at[0]],
                                          sss[b]).wait()
                pltpu.async_copy(h_hbm.at[sstage.at[j]], rows[b], gss[b])
            for b in range(2):
                j = 2 * r + b
                pltpu.make_async_copy(h_hbm.at[sstage.at[j]], rows[b],
                                      gss[b]).wait()
                pltpu.async_copy(rows[b], acc_sh.at[dstage.at[j]], sss[b],
                                 add=True)
            return 0
        lax.fori_loop(0, SCH // 2, body, 0)
        for b in range(2):
            pltpu.make_async_copy(rows[b], acc_sh.at[dstage.at[0]],
                                  sss[b]).wait()
    plsc.subcore_barrier()

    ob = c * NP + base
    pltpu.sync_copy(acc_sh.at[pl.ds(base, ARPT)],
                    out_hbm.at[pl.ds(ob, ARPT)])


def _edge_scatter(h, src2, dst2):
    k = pl.kernel(
        _scatter_body,
        out_type=jax.ShapeDtypeStruct((NC * NP, 128), jnp.float32),
        mesh=_sc_mesh(),
        scratch_types=(
            [pltpu.VMEM_SHARED((AR, 128), jnp.float32)]
            + [pltpu.VMEM((SCH, CH), jnp.int32) for _ in range(2)]
            + [pltpu.VMEM((CH, 128), jnp.float32) for _ in range(2)]
            + [pltpu.SemaphoreType.DMA for _ in range(4)]
        ),
    )
    return k(h, src2, dst2)


def _mm_scale_body(x_ref, w_ref, d0_ref, d1_ref, o_ref, dv_ref):
    dinv = lax.rsqrt(d0_ref[...] + d1_ref[...])
    dv_ref[...] = dinv
    o_ref[...] = jnp.dot(x_ref[...], w_ref[...],
                         preferred_element_type=jnp.float32) * dinv


def _mm_scale(x_p, W, deg2):
    return pl.pallas_call(
        _mm_scale_body,
        out_shape=(jax.ShapeDtypeStruct((NP, 128), jnp.float32),
                   jax.ShapeDtypeStruct((NP, 1), jnp.float32)),
        grid=(NBLK,),
        in_specs=[pl.BlockSpec((BLK, 128), lambda i: (i, 0)),
                  pl.BlockSpec((128, 128), lambda i: (0, 0)),
                  pl.BlockSpec((BLK, 1), lambda i: (i, 0)),
                  pl.BlockSpec((BLK, 1), lambda i: (i + NBLK, 0))],
        out_specs=(pl.BlockSpec((BLK, 128), lambda i: (i, 0)),
                   pl.BlockSpec((BLK, 1), lambda i: (i, 0))),
    )(x_p, W, deg2, deg2)


def _layer_mid_body(s0_ref, s1_ref, h_ref, d_ref, b_ref, w_ref, o_ref):
    i = pl.program_id(0)
    tot = s0_ref[...] + s1_ref[...] + h_ref[...]
    act = jnp.maximum(d_ref[...] * tot + b_ref[...], 0.0)
    row = i * BLK + lax.broadcasted_iota(jnp.int32, (BLK, 128), 0)
    act = jnp.where(row < N, act, 0.0)
    o_ref[...] = jnp.dot(act, w_ref[...],
                         preferred_element_type=jnp.float32) * d_ref[...]


def _layer_mid(s, ht, dinv2, b1r, W2):
    return pl.pallas_call(
        _layer_mid_body,
        out_shape=jax.ShapeDtypeStruct((NP, 128), jnp.float32),
        grid=(NBLK,),
        in_specs=[pl.BlockSpec((BLK, 128), lambda i: (i, 0)),
                  pl.BlockSpec((BLK, 128), lambda i: (i + NBLK, 0)),
                  pl.BlockSpec((BLK, 128), lambda i: (i, 0)),
                  pl.BlockSpec((BLK, 1), lambda i: (i, 0)),
                  pl.BlockSpec((1, 128), lambda i: (0, 0)),
                  pl.BlockSpec((128, 128), lambda i: (0, 0))],
        out_specs=pl.BlockSpec((BLK, 128), lambda i: (i, 0)),
    )(s, s, ht, dinv2, b1r, W2)


def _finale_body(s0_ref, s1_ref, h_ref, d_ref, b_ref, bt_ref, c0_ref, c1_ref,
                 wl_ref, bl_ref, wd1_ref, bd1_ref, wd2_ref, bd2_ref, o_ref):
    i = pl.program_id(0)
    tot = s0_ref[...] + s1_ref[...] + h_ref[...]
    act = jnp.maximum(d_ref[...] * tot + b_ref[...], 0.0)
    row = i * BLK + lax.broadcasted_iota(jnp.int32, (BLK, 128), 0)
    act = jnp.where(row < N, act, 0.0)
    gid = lax.broadcasted_iota(jnp.int32, (BLK, G), 1)
    onehot = jnp.where(bt_ref[...].astype(jnp.int32) == gid, 1.0, 0.0)
    part = lax.dot_general(onehot, act, (((0,), (0,)), ((), ())),
                           preferred_element_type=jnp.float32)

    @pl.when(i == 0)
    def _():
        o_ref[...] = jnp.zeros_like(o_ref)

    o_ref[...] += part

    @pl.when(i == NBLK - 1)
    def _():
        pooled = o_ref[...] / jnp.maximum(c0_ref[...] + c1_ref[...], 1.0)
        lab = jax.nn.sigmoid(
            jnp.dot(pooled, wl_ref[...], preferred_element_type=jnp.float32)
            + bl_ref[...])
        dmid = jnp.maximum(
            jnp.dot(pooled, wd1_ref[...], preferred_element_type=jnp.float32)
            + bd1_ref[...], 0.0)
        dom = (jnp.dot(dmid, wd2_ref[...], preferred_element_type=jnp.float32)
               + bd2_ref[...])
        col = lax.broadcasted_iota(jnp.int32, (G, 128), 1)
        o_ref[...] = jnp.where(col == 0, lab, dom)


def _finale(s, ht, dinv2, b2r, batchf, cnt2, Wlp, blp, Wd1, bd1r, Wd2p, bd2p):
    return pl.pallas_call(
        _finale_body,
        out_shape=jax.ShapeDtypeStruct((G, 128), jnp.float32),
        grid=(NBLK,),
        in_specs=[pl.BlockSpec((BLK, 128), lambda i: (i, 0)),
                  pl.BlockSpec((BLK, 128), lambda i: (i + NBLK, 0)),
                  pl.BlockSpec((BLK, 128), lambda i: (i, 0)),
                  pl.BlockSpec((BLK, 1), lambda i: (i, 0)),
                  pl.BlockSpec((1, 128), lambda i: (0, 0)),
                  pl.BlockSpec((BLK, 1), lambda i: (i, 0)),
                  pl.BlockSpec((G, 1), lambda i: (0, 0)),
                  pl.BlockSpec((G, 1), lambda i: (1, 0)),
                  pl.BlockSpec((128, 128), lambda i: (0, 0)),
                  pl.BlockSpec((1, 128), lambda i: (0, 0)),
                  pl.BlockSpec((128, 64), lambda i: (0, 0)),
                  pl.BlockSpec((1, 64), lambda i: (0, 0)),
                  pl.BlockSpec((64, 128), lambda i: (0, 0)),
                  pl.BlockSpec((1, 128), lambda i: (0, 0))],
        out_specs=pl.BlockSpec((G, 128), lambda i: (0, 0)),
    )(s, s, ht, dinv2, b2r, batchf, cnt2, cnt2, Wlp, blp, Wd1, bd1r, Wd2p, bd2p)


def kernel(x, edge_index, batch, W1, b1, W2, b2, Wl, bl, Wd1, bd1, Wd2, bd2):
    npad = NP - N
    epad = ECH * CH - E
    ar = jnp.arange(epad, dtype=jnp.int32)
    src2 = jnp.concatenate([edge_index[0], N + ar % npad]).reshape(ECH, CH)
    dst2 = jnp.concatenate([edge_index[1], N + ar % (AR - N)]).reshape(ECH, CH)
    batch_p = jnp.concatenate([batch, jnp.full((npad,), G, jnp.int32)])

    deg, cnt = _deg(dst2, batch_p)
    h1t, dinv2 = _mm_scale(x, W1, deg.reshape(NC * NP, 1))
    s1 = _edge_scatter(h1t, src2, dst2)
    h2t = _layer_mid(s1, h1t, dinv2, b1[None, :], W2)
    s2 = _edge_scatter(h2t, src2, dst2)

    heads = _finale(
        s2, h2t, dinv2, b2[None, :],
        batch_p.astype(jnp.float32).reshape(NP, 1), cnt.reshape(NC * G, 1),
        jnp.pad(Wl, ((0, 0), (0, 127))), jnp.pad(bl[None, :], ((0, 0), (0, 127))),
        Wd1, bd1[None, :],
        jnp.pad(Wd2, ((0, 0), (1, 125))), jnp.pad(bd2[None, :], ((0, 0), (1, 125))),
    )
    return heads[:, 0:1], heads[:, 1:3]
